# baseline (speedup 1.0000x reference)
import functools

import jax
import jax.numpy as jnp
from jax import lax
from jax.experimental import pallas as pl
from jax.experimental.pallas import tpu as pltpu
from jax.experimental.pallas import tpu_sc as plsc

N = 10000
E = 320000
AMB = 129
HW = 80
NC = 2
NS = 16
EPT = E // NS
K = 128
CH = 158
PAD = CH * K - EPT
GR = 16
RPT = N // NS
ZR = 25
VPR = HW // 16


def _time_col(v):
    return jnp.sqrt(1.0 + jnp.sum(v * v, axis=-1, keepdims=True))


def _split_halves(h):
    top = h[:, :HW]
    bot = jnp.concatenate(
        [h[:, HW:], jnp.ones((N, 1), jnp.float32),
         jnp.zeros((N, 2 * HW - AMB - 1), jnp.float32)], axis=1)
    return jnp.concatenate([top, bot], axis=0)


def _combine_halves(acc):
    num = jnp.concatenate([acc[0], acc[1][:, :AMB - HW]], axis=1)
    den = acc[1][:, AMB - HW:AMB - HW + 1]
    return num, den


def _centroid_space(num, den):
    mu = num / (den + 1e-16)
    q = jnp.clip(mu[:, :1] * mu[:, :1]
                 - jnp.sum(mu[:, 1:] * mu[:, 1:], axis=1, keepdims=True),
                 1e-8, None)
    return mu[:, 1:] / jnp.sqrt(q)


def _dense_in_body(x_ref, w_ref, as_ref, ad_ref, hx_ref, s_ref, d_ref):
    x = x_ref[...]
    v = lax.dot_general(x, w_ref[...], (((1,), (1,)), ((), ())),
                        preferred_element_type=jnp.float32)
    h = jnp.concatenate([_time_col(v), v], axis=1)
    s_ref[...] = jnp.dot(h, as_ref[...])
    d_ref[...] = jnp.dot(h, ad_ref[...])
    hx_ref[...] = _split_halves(h)


def _mid_body(acc_ref, w_ref, as_ref, ad_ref, hx_ref, s_ref, d_ref):
    num, den = _combine_halves(acc_ref[...])
    space = _centroid_space(num, den)
    g = jax.nn.gelu(space)
    h1 = jnp.concatenate([_time_col(g), g], axis=1)
    v = lax.dot_general(h1, w_ref[...], (((1,), (1,)), ((), ())),
                        preferred_element_type=jnp.float32)
    h2 = jnp.concatenate([_time_col(v), v], axis=1)
    s_ref[...] = jnp.dot(h2, as_ref[...])
    d_ref[...] = jnp.dot(h2, ad_ref[...])
    hx_ref[...] = _split_halves(h2)


def _fin_body(acc_ref, w_ref, out_ref, gm_ref):
    num, den = _combine_halves(acc_ref[...])
    space = _centroid_space(num, den)
    h2 = jnp.concatenate([_time_col(space), space], axis=1)
    xr = h2.reshape(8, N // 8, AMB)
    mean = jnp.mean(xr, axis=1)
    q2 = jnp.clip(mean[:, :1] * mean[:, :1]
                  - jnp.sum(mean[:, 1:] * mean[:, 1:], axis=1, keepdims=True),
                  1e-8, None)
    gm_ref[...] = mean / jnp.sqrt(q2)
    xf = xr[:, 0, :]
    v = lax.dot_general(xf, w_ref[...], (((1,), (1,)), ((), ())),
                        preferred_element_type=jnp.float32)
    out_ref[...] = jnp.concatenate([_time_col(v), v], axis=1)


def _edge_body(hx_hbm, s_hbm, d_hbm, pk_hbm, acc_out,
               pk_all, gsrc_a, gsrc_b, dst_a, dst_b, s_v, d_v,
               rows_a, rows_b, zrow_v, acc_sh,
               sem_ga, sem_gb, sem_sa, sem_sb):
    cid = lax.axis_index("c")
    sid = lax.axis_index("s")

    zvec = jnp.zeros((16,), jnp.float32)
    for i in range(ZR):
        for j in range(VPR):
            zrow_v[i, pl.ds(j * 16, 16)] = zvec

    def zero_body(t, _):
        r0 = pl.multiple_of(sid * RPT + t * ZR, ZR)
        pltpu.sync_copy(zrow_v, acc_sh.at[pl.ds(r0, ZR)])
        return _

    lax.fori_loop(0, RPT // ZR, zero_body, None)
    @pl.when(sid == 0)
    def _():
        pltpu.sync_copy(zrow_v.at[pl.ds(0, GR)], acc_sh.at[pl.ds(N, GR)])

    pltpu.sync_copy(s_hbm, s_v)
    pltpu.sync_copy(d_hbm, d_v)
    pltpu.sync_copy(pk_hbm.at[sid], pk_all)
    plsc.subcore_barrier()

    goff = jnp.full((16,), cid * N, jnp.int32)
    lomask = jnp.full((16,), 0xFFFF, jnp.int32)

    def build_gidx(c, gsrcr, dstr):
        for g in range(K // 16):
            pk = pk_all[c, pl.ds(g * 16, 16)]
            gsrcr[pl.ds(g * 16, 16)] = (pk & lomask) + goff
            dstr[pl.ds(g * 16, 16)] = pk >> 16

    def vals_of(c, dstr):
        vals = []
        for g in range(K // 16):
            src = pk_all[c, pl.ds(g * 16, 16)] & lomask
            sg = plsc.load_gather(s_v, [src])
            dg = plsc.load_gather(d_v, [dstr[pl.ds(g * 16, 16)]])
            a = sg + dg
            vals.append(jnp.exp(jnp.maximum(a, 0.2 * a)))
        return vals

    def scale(rowsr, vals):
        for g in range(K // 16):
            val = vals[g]
            for r in range(16):
                vsplat = jnp.full((16,), val[r], jnp.float32)
                row = g * 16 + r
                for j in range(VPR):
                    rowsr[row, pl.ds(j * 16, 16)] = (
                        rowsr[row, pl.ds(j * 16, 16)] * vsplat)

    def pair_body(i, _):
        ca = 2 * i
        cb = 2 * i + 1
        build_gidx(ca, gsrc_a, dst_a)
        ga = pltpu.async_copy(hx_hbm.at[gsrc_a], rows_a, sem_ga)
        build_gidx(cb, gsrc_b, dst_b)
        gb = pltpu.async_copy(hx_hbm.at[gsrc_b], rows_b, sem_gb)
        vals_a = vals_of(ca, dst_a)
        ga.wait()
        scale(rows_a, vals_a)
        sa = pltpu.make_async_copy(rows_a, acc_sh.at[dst_a], sem_sa)
        sa.start(add=True)
        vals_b = vals_of(cb, dst_b)
        gb.wait()
        scale(rows_b, vals_b)
        sb = pltpu.make_async_copy(rows_b, acc_sh.at[dst_b], sem_sb)
        sb.start(add=True)
        sa.wait()
        sb.wait()
        return _

    lax.fori_loop(0, CH // 2, pair_body, None)

    plsc.subcore_barrier()
    r0 = sid * RPT
    pltpu.sync_copy(acc_sh.at[pl.ds(r0, RPT)], acc_out.at[cid, pl.ds(r0, RPT)])


_edge_kernel = functools.partial(
    pl.kernel,
    out_type=jax.ShapeDtypeStruct((NC, N, HW), jnp.float32),
    mesh=plsc.VectorSubcoreMesh(core_axis_name="c", subcore_axis_name="s"),
    compiler_params=pltpu.CompilerParams(use_tc_tiling_on_sc=False,
                                         needs_layout_passes=False),
    scratch_types=[
        pltpu.VMEM((CH, K), jnp.int32),
        pltpu.VMEM((K,), jnp.int32),
        pltpu.VMEM((K,), jnp.int32),
        pltpu.VMEM((K,), jnp.int32),
        pltpu.VMEM((K,), jnp.int32),
        pltpu.VMEM((N,), jnp.float32),
        pltpu.VMEM((N,), jnp.float32),
        pltpu.VMEM((K, HW), jnp.float32),
        pltpu.VMEM((K, HW), jnp.float32),
        pltpu.VMEM((ZR, HW), jnp.float32),
        pltpu.VMEM_SHARED((N + GR, HW), jnp.float32),
        pltpu.SemaphoreType.DMA,
        pltpu.SemaphoreType.DMA,
        pltpu.SemaphoreType.DMA,
        pltpu.SemaphoreType.DMA,
    ],
)(_edge_body)


def _dense_in(x, W1, a1s, a1d):
    return pl.pallas_call(
        _dense_in_body,
        out_shape=[
            jax.ShapeDtypeStruct((2 * N, HW), jnp.float32),
            jax.ShapeDtypeStruct((N, 1), jnp.float32),
            jax.ShapeDtypeStruct((N, 1), jnp.float32),
        ],
    )(x, W1, a1s, a1d)


def _mid(acc, W2, a2s, a2d):
    return pl.pallas_call(
        _mid_body,
        out_shape=[
            jax.ShapeDtypeStruct((2 * N, HW), jnp.float32),
            jax.ShapeDtypeStruct((N, 1), jnp.float32),
            jax.ShapeDtypeStruct((N, 1), jnp.float32),
        ],
    )(acc, W2, a2s, a2d)


def _fin(acc, W3):
    return pl.pallas_call(
        _fin_body,
        out_shape=[
            jax.ShapeDtypeStruct((8, AMB), jnp.float32),
            jax.ShapeDtypeStruct((8, AMB), jnp.float32),
        ],
    )(acc, W3)


def _pad_edges(edge_index):
    src = edge_index[0].reshape(NS, EPT)
    dst = edge_index[1].reshape(NS, EPT)
    pad_src = jnp.zeros((NS, PAD), jnp.int32)
    pad_dst = jnp.tile(N + (jnp.arange(PAD, dtype=jnp.int32) % GR), (NS, 1))
    srcp = jnp.concatenate([src, pad_src], axis=1)
    dstp = jnp.concatenate([dst, pad_dst], axis=1)
    return ((dstp << 16) | srcp).reshape(NS, CH, K)


def kernel(x, edge_index, W1, a1_src, a1_dst, W2, a2_src, a2_dst, W3, batch_size):
    a1s = a1_src.reshape(AMB, 1)
    a1d = a1_dst.reshape(AMB, 1)
    a2s = a2_src.reshape(AMB, 1)
    a2d = a2_dst.reshape(AMB, 1)
    pk = _pad_edges(edge_index)

    hx1, s1, d1 = _dense_in(x, W1, a1s, a1d)
    acc1 = _edge_kernel(hx1, s1.reshape(N), d1.reshape(N), pk)
    hx2, s2, d2 = _mid(acc1, W2, a2s, a2d)
    acc2 = _edge_kernel(hx2, s2.reshape(N), d2.reshape(N), pk)
    out, gm = _fin(acc2, W3)
    return (out, gm)


# speedup vs baseline: 1.0035x; 1.0035x over previous
import functools

import jax
import jax.numpy as jnp
from jax import lax
from jax.experimental import pallas as pl
from jax.experimental.pallas import tpu as pltpu
from jax.experimental.pallas import tpu_sc as plsc

N = 10000
E = 320000
AMB = 129
HW = 80
NC = 2
NS = 16
EPT = E // NS
K = 128
CH = 158
PAD = CH * K - EPT
GR = 16
RPT = N // NS
ZR = 25
VPR = HW // 16


def _time_col(v):
    return jnp.sqrt(1.0 + jnp.sum(v * v, axis=-1, keepdims=True))


def _split_halves(h):
    top = h[:, :HW]
    bot = jnp.concatenate(
        [h[:, HW:], jnp.ones((N, 1), jnp.float32),
         jnp.zeros((N, 2 * HW - AMB - 1), jnp.float32)], axis=1)
    return jnp.concatenate([top, bot], axis=0)


def _combine_halves(acc):
    num = jnp.concatenate([acc[0], acc[1][:, :AMB - HW]], axis=1)
    den = acc[1][:, AMB - HW:AMB - HW + 1]
    return num, den


def _centroid_space(num, den):
    mu = num / (den + 1e-16)
    q = jnp.clip(mu[:, :1] * mu[:, :1]
                 - jnp.sum(mu[:, 1:] * mu[:, 1:], axis=1, keepdims=True),
                 1e-8, None)
    return mu[:, 1:] / jnp.sqrt(q)


def _dense_in_body(x_ref, w_ref, as_ref, ad_ref, hx_ref, s_ref, d_ref):
    x = x_ref[...]
    v = lax.dot_general(x, w_ref[...], (((1,), (1,)), ((), ())),
                        preferred_element_type=jnp.float32)
    h = jnp.concatenate([_time_col(v), v], axis=1)
    s_ref[...] = jnp.dot(h, as_ref[...])
    d_ref[...] = jnp.dot(h, ad_ref[...])
    hx_ref[...] = _split_halves(h)


def _mid_body(acc_ref, w_ref, as_ref, ad_ref, hx_ref, s_ref, d_ref):
    num, den = _combine_halves(acc_ref[...])
    space = _centroid_space(num, den)
    g = jax.nn.gelu(space)
    h1 = jnp.concatenate([_time_col(g), g], axis=1)
    v = lax.dot_general(h1, w_ref[...], (((1,), (1,)), ((), ())),
                        preferred_element_type=jnp.float32)
    h2 = jnp.concatenate([_time_col(v), v], axis=1)
    s_ref[...] = jnp.dot(h2, as_ref[...])
    d_ref[...] = jnp.dot(h2, ad_ref[...])
    hx_ref[...] = _split_halves(h2)


def _fin_body(acc_ref, w_ref, out_ref, gm_ref):
    num, den = _combine_halves(acc_ref[...])
    space = _centroid_space(num, den)
    h2 = jnp.concatenate([_time_col(space), space], axis=1)
    xr = h2.reshape(8, N // 8, AMB)
    mean = jnp.mean(xr, axis=1)
    q2 = jnp.clip(mean[:, :1] * mean[:, :1]
                  - jnp.sum(mean[:, 1:] * mean[:, 1:], axis=1, keepdims=True),
                  1e-8, None)
    gm_ref[...] = mean / jnp.sqrt(q2)
    xf = xr[:, 0, :]
    v = lax.dot_general(xf, w_ref[...], (((1,), (1,)), ((), ())),
                        preferred_element_type=jnp.float32)
    out_ref[...] = jnp.concatenate([_time_col(v), v], axis=1)


def _edge_body(hx_hbm, s_hbm, d_hbm, pk_hbm, acc_out,
               pk_all, gsrc_a, gsrc_b, dst_a, dst_b, s_v, d_v,
               rows_a, rows_b, zrow_v, acc_sh,
               sem_ga, sem_gb, sem_sa, sem_sb):
    cid = lax.axis_index("c")
    sid = lax.axis_index("s")

    zvec = jnp.zeros((16,), jnp.float32)
    for i in range(ZR):
        for j in range(VPR):
            zrow_v[i, pl.ds(j * 16, 16)] = zvec

    def zero_body(t, _):
        r0 = pl.multiple_of(sid * RPT + t * ZR, ZR)
        pltpu.sync_copy(zrow_v, acc_sh.at[pl.ds(r0, ZR)])
        return _

    lax.fori_loop(0, RPT // ZR, zero_body, None)
    @pl.when(sid == 0)
    def _():
        pltpu.sync_copy(zrow_v.at[pl.ds(0, GR)], acc_sh.at[pl.ds(N, GR)])

    pltpu.sync_copy(s_hbm, s_v)
    pltpu.sync_copy(d_hbm, d_v)
    pltpu.sync_copy(pk_hbm.at[sid], pk_all)
    plsc.subcore_barrier()

    goff = jnp.full((16,), cid * N, jnp.int32)
    lomask = jnp.full((16,), 0xFFFF, jnp.int32)

    def build_gidx(c, gsrcr, dstr):
        for g in range(K // 16):
            pk = pk_all[c, pl.ds(g * 16, 16)]
            gsrcr[pl.ds(g * 16, 16)] = (pk & lomask) + goff
            dstr[pl.ds(g * 16, 16)] = pk >> 16

    def vals_of(c, dstr):
        vals = []
        for g in range(K // 16):
            src = pk_all[c, pl.ds(g * 16, 16)] & lomask
            sg = plsc.load_gather(s_v, [src])
            dg = plsc.load_gather(d_v, [dstr[pl.ds(g * 16, 16)]])
            a = sg + dg
            vals.append(jnp.exp(jnp.maximum(a, 0.2 * a)))
        return vals

    def scale(rowsr, vals):
        for g in range(K // 16):
            val = vals[g]
            for r in range(16):
                vsplat = jnp.full((16,), val[r], jnp.float32)
                row = g * 16 + r
                for j in range(VPR):
                    rowsr[row, pl.ds(j * 16, 16)] = (
                        rowsr[row, pl.ds(j * 16, 16)] * vsplat)

    def drain_scatters():
        pltpu.make_async_copy(hx_hbm.at[gsrc_a], rows_a, sem_sa).wait()
        pltpu.make_async_copy(hx_hbm.at[gsrc_b], rows_b, sem_sb).wait()

    def pair_body(i, _):
        ca = 2 * i
        cb = 2 * i + 1

        @pl.when(i > 0)
        def _():
            drain_scatters()

        build_gidx(ca, gsrc_a, dst_a)
        ga = pltpu.async_copy(hx_hbm.at[gsrc_a], rows_a, sem_ga)
        build_gidx(cb, gsrc_b, dst_b)
        gb = pltpu.async_copy(hx_hbm.at[gsrc_b], rows_b, sem_gb)
        vals_a = vals_of(ca, dst_a)
        vals_b = vals_of(cb, dst_b)
        ga.wait()
        scale(rows_a, vals_a)
        pltpu.make_async_copy(rows_a, acc_sh.at[dst_a], sem_sa).start(add=True)
        gb.wait()
        scale(rows_b, vals_b)
        pltpu.make_async_copy(rows_b, acc_sh.at[dst_b], sem_sb).start(add=True)
        return _

    lax.fori_loop(0, CH // 2, pair_body, None)
    drain_scatters()

    plsc.subcore_barrier()
    r0 = sid * RPT
    pltpu.sync_copy(acc_sh.at[pl.ds(r0, RPT)], acc_out.at[cid, pl.ds(r0, RPT)])


_edge_kernel = functools.partial(
    pl.kernel,
    out_type=jax.ShapeDtypeStruct((NC, N, HW), jnp.float32),
    mesh=plsc.VectorSubcoreMesh(core_axis_name="c", subcore_axis_name="s"),
    compiler_params=pltpu.CompilerParams(use_tc_tiling_on_sc=False,
                                         needs_layout_passes=False),
    scratch_types=[
        pltpu.VMEM((CH, K), jnp.int32),
        pltpu.VMEM((K,), jnp.int32),
        pltpu.VMEM((K,), jnp.int32),
        pltpu.VMEM((K,), jnp.int32),
        pltpu.VMEM((K,), jnp.int32),
        pltpu.VMEM((N,), jnp.float32),
        pltpu.VMEM((N,), jnp.float32),
        pltpu.VMEM((K, HW), jnp.float32),
        pltpu.VMEM((K, HW), jnp.float32),
        pltpu.VMEM((ZR, HW), jnp.float32),
        pltpu.VMEM_SHARED((N + GR, HW), jnp.float32),
        pltpu.SemaphoreType.DMA,
        pltpu.SemaphoreType.DMA,
        pltpu.SemaphoreType.DMA,
        pltpu.SemaphoreType.DMA,
    ],
)(_edge_body)


def _dense_in(x, W1, a1s, a1d):
    return pl.pallas_call(
        _dense_in_body,
        out_shape=[
            jax.ShapeDtypeStruct((2 * N, HW), jnp.float32),
            jax.ShapeDtypeStruct((N, 1), jnp.float32),
            jax.ShapeDtypeStruct((N, 1), jnp.float32),
        ],
    )(x, W1, a1s, a1d)


def _mid(acc, W2, a2s, a2d):
    return pl.pallas_call(
        _mid_body,
        out_shape=[
            jax.ShapeDtypeStruct((2 * N, HW), jnp.float32),
            jax.ShapeDtypeStruct((N, 1), jnp.float32),
            jax.ShapeDtypeStruct((N, 1), jnp.float32),
        ],
    )(acc, W2, a2s, a2d)


def _fin(acc, W3):
    return pl.pallas_call(
        _fin_body,
        out_shape=[
            jax.ShapeDtypeStruct((8, AMB), jnp.float32),
            jax.ShapeDtypeStruct((8, AMB), jnp.float32),
        ],
    )(acc, W3)


def _pad_edges(edge_index):
    src = edge_index[0].reshape(NS, EPT)
    dst = edge_index[1].reshape(NS, EPT)
    pad_src = jnp.zeros((NS, PAD), jnp.int32)
    pad_dst = jnp.tile(N + (jnp.arange(PAD, dtype=jnp.int32) % GR), (NS, 1))
    srcp = jnp.concatenate([src, pad_src], axis=1)
    dstp = jnp.concatenate([dst, pad_dst], axis=1)
    return ((dstp << 16) | srcp).reshape(NS, CH, K)


def kernel(x, edge_index, W1, a1_src, a1_dst, W2, a2_src, a2_dst, W3, batch_size):
    a1s = a1_src.reshape(AMB, 1)
    a1d = a1_dst.reshape(AMB, 1)
    a2s = a2_src.reshape(AMB, 1)
    a2d = a2_dst.reshape(AMB, 1)
    pk = _pad_edges(edge_index)

    hx1, s1, d1 = _dense_in(x, W1, a1s, a1d)
    acc1 = _edge_kernel(hx1, s1.reshape(N), d1.reshape(N), pk)
    hx2, s2, d2 = _mid(acc1, W2, a2s, a2d)
    acc2 = _edge_kernel(hx2, s2.reshape(N), d2.reshape(N), pk)
    out, gm = _fin(acc2, W3)
    return (out, gm)


# speedup vs baseline: 1.0794x; 1.0757x over previous
import functools

import jax
import jax.numpy as jnp
from jax import lax
from jax.experimental import pallas as pl
from jax.experimental.pallas import tpu as pltpu
from jax.experimental.pallas import tpu_sc as plsc

N = 10000
E = 320000
AMB = 129
HW = 80
NC = 2
NS = 16
EPT = E // NS
K = 128
CH = 158
PAD = CH * K - EPT
GR = 16
RPT = N // NS
ZR = 25
VPR = HW // 16


def _time_col(v):
    return jnp.sqrt(1.0 + jnp.sum(v * v, axis=-1, keepdims=True))


def _split_halves(h):
    top = h[:, :HW]
    bot = jnp.concatenate(
        [h[:, HW:], jnp.ones((N, 1), jnp.float32),
         jnp.zeros((N, 2 * HW - AMB - 1), jnp.float32)], axis=1)
    return jnp.concatenate([top, bot], axis=0)


def _combine_halves(acc):
    num = jnp.concatenate([acc[0], acc[1][:, :AMB - HW]], axis=1)
    den = acc[1][:, AMB - HW:AMB - HW + 1]
    return num, den


def _centroid_space(num, den):
    mu = num / (den + 1e-16)
    q = jnp.clip(mu[:, :1] * mu[:, :1]
                 - jnp.sum(mu[:, 1:] * mu[:, 1:], axis=1, keepdims=True),
                 1e-8, None)
    return mu[:, 1:] / jnp.sqrt(q)


def _dense_in_body(x_ref, w_ref, as_ref, ad_ref, hx_ref, s_ref, d_ref):
    x = x_ref[...]
    v = lax.dot_general(x, w_ref[...], (((1,), (1,)), ((), ())),
                        preferred_element_type=jnp.float32)
    h = jnp.concatenate([_time_col(v), v], axis=1)
    s_ref[...] = jnp.dot(h, as_ref[...])
    d_ref[...] = jnp.dot(h, ad_ref[...])
    hx_ref[...] = _split_halves(h)


def _mid_body(acc_ref, w_ref, as_ref, ad_ref, hx_ref, s_ref, d_ref):
    num, den = _combine_halves(acc_ref[...])
    space = _centroid_space(num, den)
    g = jax.nn.gelu(space)
    h1 = jnp.concatenate([_time_col(g), g], axis=1)
    v = lax.dot_general(h1, w_ref[...], (((1,), (1,)), ((), ())),
                        preferred_element_type=jnp.float32)
    h2 = jnp.concatenate([_time_col(v), v], axis=1)
    s_ref[...] = jnp.dot(h2, as_ref[...])
    d_ref[...] = jnp.dot(h2, ad_ref[...])
    hx_ref[...] = _split_halves(h2)


def _fin_body(acc_ref, w_ref, out_ref, gm_ref):
    num, den = _combine_halves(acc_ref[...])
    space = _centroid_space(num, den)
    h2 = jnp.concatenate([_time_col(space), space], axis=1)
    xr = h2.reshape(8, N // 8, AMB)
    mean = jnp.mean(xr, axis=1)
    q2 = jnp.clip(mean[:, :1] * mean[:, :1]
                  - jnp.sum(mean[:, 1:] * mean[:, 1:], axis=1, keepdims=True),
                  1e-8, None)
    gm_ref[...] = mean / jnp.sqrt(q2)
    xf = xr[:, 0, :]
    v = lax.dot_general(xf, w_ref[...], (((1,), (1,)), ((), ())),
                        preferred_element_type=jnp.float32)
    out_ref[...] = jnp.concatenate([_time_col(v), v], axis=1)


def _edge_body(hx_hbm, s_hbm, d_hbm, pk_hbm, acc_out,
               pk_all, gsrc_a, gsrc_b, dst_a, dst_b, s_v, d_v,
               rows_a, rows_b, zrow_v, acc_sh,
               sem_ga, sem_gb, sem_sa, sem_sb):
    cid = lax.axis_index("c")
    sid = lax.axis_index("s")

    zvec = jnp.zeros((16,), jnp.float32)
    for i in range(ZR):
        for j in range(VPR):
            zrow_v[i, pl.ds(j * 16, 16)] = zvec

    def zero_body(t, _):
        r0 = pl.multiple_of(sid * RPT + t * ZR, ZR)
        pltpu.sync_copy(zrow_v, acc_sh.at[pl.ds(r0, ZR)])
        return _

    lax.fori_loop(0, RPT // ZR, zero_body, None)
    @pl.when(sid == 0)
    def _():
        pltpu.sync_copy(zrow_v.at[pl.ds(0, GR)], acc_sh.at[pl.ds(N, GR)])

    pltpu.sync_copy(s_hbm, s_v)
    pltpu.sync_copy(d_hbm, d_v)
    pltpu.sync_copy(pk_hbm.at[sid], pk_all)
    plsc.subcore_barrier()

    goff = jnp.full((16,), cid * N, jnp.int32)
    lomask = jnp.full((16,), 0xFFFF, jnp.int32)

    def build_gidx(c, gsrcr, dstr):
        for g in range(K // 16):
            pk = pk_all[c, pl.ds(g * 16, 16)]
            gsrcr[pl.ds(g * 16, 16)] = (pk & lomask) + goff
            dstr[pl.ds(g * 16, 16)] = pk >> 16

    def vals_of(c):
        vals = []
        for g in range(K // 16):
            pk = pk_all[c, pl.ds(g * 16, 16)]
            sg = plsc.load_gather(s_v, [pk & lomask])
            dg = plsc.load_gather(d_v, [pk >> 16])
            a = sg + dg
            vals.append(jnp.exp(jnp.maximum(a, 0.2 * a)))
        return vals

    def scale(rowsr, vals):
        for g in range(K // 16):
            val = vals[g]
            for r in range(16):
                vsplat = jnp.full((16,), val[r], jnp.float32)
                row = g * 16 + r
                for j in range(VPR):
                    rowsr[row, pl.ds(j * 16, 16)] = (
                        rowsr[row, pl.ds(j * 16, 16)] * vsplat)

    def drain_a():
        pltpu.make_async_copy(hx_hbm.at[gsrc_a], rows_a, sem_sa).wait()

    def drain_b():
        pltpu.make_async_copy(hx_hbm.at[gsrc_b], rows_b, sem_sb).wait()

    def pair_body(i, _):
        ca = 2 * i
        cb = 2 * i + 1

        @pl.when(i > 0)
        def _():
            drain_a()

        build_gidx(ca, gsrc_a, dst_a)
        ga = pltpu.async_copy(hx_hbm.at[gsrc_a], rows_a, sem_ga)
        vals_a = vals_of(ca)
        vals_b = vals_of(cb)

        @pl.when(i > 0)
        def _():
            drain_b()

        build_gidx(cb, gsrc_b, dst_b)
        gb = pltpu.async_copy(hx_hbm.at[gsrc_b], rows_b, sem_gb)
        ga.wait()
        scale(rows_a, vals_a)
        pltpu.make_async_copy(rows_a, acc_sh.at[dst_a], sem_sa).start(add=True)
        gb.wait()
        scale(rows_b, vals_b)
        pltpu.make_async_copy(rows_b, acc_sh.at[dst_b], sem_sb).start(add=True)
        return _

    lax.fori_loop(0, CH // 2, pair_body, None)
    drain_a()
    drain_b()

    plsc.subcore_barrier()
    r0 = sid * RPT
    pltpu.sync_copy(acc_sh.at[pl.ds(r0, RPT)], acc_out.at[cid, pl.ds(r0, RPT)])


_edge_kernel = functools.partial(
    pl.kernel,
    out_type=jax.ShapeDtypeStruct((NC, N, HW), jnp.float32),
    mesh=plsc.VectorSubcoreMesh(core_axis_name="c", subcore_axis_name="s"),
    compiler_params=pltpu.CompilerParams(use_tc_tiling_on_sc=False,
                                         needs_layout_passes=False),
    scratch_types=[
        pltpu.VMEM((CH, K), jnp.int32),
        pltpu.VMEM((K,), jnp.int32),
        pltpu.VMEM((K,), jnp.int32),
        pltpu.VMEM((K,), jnp.int32),
        pltpu.VMEM((K,), jnp.int32),
        pltpu.VMEM((N,), jnp.float32),
        pltpu.VMEM((N,), jnp.float32),
        pltpu.VMEM((K, HW), jnp.float32),
        pltpu.VMEM((K, HW), jnp.float32),
        pltpu.VMEM((ZR, HW), jnp.float32),
        pltpu.VMEM_SHARED((N + GR, HW), jnp.float32),
        pltpu.SemaphoreType.DMA,
        pltpu.SemaphoreType.DMA,
        pltpu.SemaphoreType.DMA,
        pltpu.SemaphoreType.DMA,
    ],
)(_edge_body)


def _dense_in(x, W1, a1s, a1d):
    return pl.pallas_call(
        _dense_in_body,
        out_shape=[
            jax.ShapeDtypeStruct((2 * N, HW), jnp.float32),
            jax.ShapeDtypeStruct((N, 1), jnp.float32),
            jax.ShapeDtypeStruct((N, 1), jnp.float32),
        ],
    )(x, W1, a1s, a1d)


def _mid(acc, W2, a2s, a2d):
    return pl.pallas_call(
        _mid_body,
        out_shape=[
            jax.ShapeDtypeStruct((2 * N, HW), jnp.float32),
            jax.ShapeDtypeStruct((N, 1), jnp.float32),
            jax.ShapeDtypeStruct((N, 1), jnp.float32),
        ],
    )(acc, W2, a2s, a2d)


def _fin(acc, W3):
    return pl.pallas_call(
        _fin_body,
        out_shape=[
            jax.ShapeDtypeStruct((8, AMB), jnp.float32),
            jax.ShapeDtypeStruct((8, AMB), jnp.float32),
        ],
    )(acc, W3)


def _pad_edges(edge_index):
    src = edge_index[0].reshape(NS, EPT)
    dst = edge_index[1].reshape(NS, EPT)
    pad_src = jnp.zeros((NS, PAD), jnp.int32)
    pad_dst = jnp.tile(N + (jnp.arange(PAD, dtype=jnp.int32) % GR), (NS, 1))
    srcp = jnp.concatenate([src, pad_src], axis=1)
    dstp = jnp.concatenate([dst, pad_dst], axis=1)
    return ((dstp << 16) | srcp).reshape(NS, CH, K)


def kernel(x, edge_index, W1, a1_src, a1_dst, W2, a2_src, a2_dst, W3, batch_size):
    a1s = a1_src.reshape(AMB, 1)
    a1d = a1_dst.reshape(AMB, 1)
    a2s = a2_src.reshape(AMB, 1)
    a2d = a2_dst.reshape(AMB, 1)
    pk = _pad_edges(edge_index)

    hx1, s1, d1 = _dense_in(x, W1, a1s, a1d)
    acc1 = _edge_kernel(hx1, s1.reshape(N), d1.reshape(N), pk)
    hx2, s2, d2 = _mid(acc1, W2, a2s, a2d)
    acc2 = _edge_kernel(hx2, s2.reshape(N), d2.reshape(N), pk)
    out, gm = _fin(acc2, W3)
    return (out, gm)
